# baseline (speedup 1.0000x reference)
import functools

import jax
import jax.numpy as jnp
from jax import lax
from jax.experimental import pallas as pl
from jax.experimental.pallas import tpu as pltpu
from jax.experimental.pallas import tpu_sc as plsc

_D = _H = _W = 128
_C = 2
_B = 2
_VPB = _D * _H * _W
_NW = 32
_TPB = 16
_VPT = _VPB // _TPB
_CH = 128
_NCHUNK = _VPT // _CH


def _dim_corners(r):
    ti = r.astype(jnp.int32)
    tf = ti.astype(jnp.float32)
    f0 = ti - jnp.where(tf > r, 1, 0)
    c0 = jnp.minimum(jnp.maximum(f0, 0), 127)
    c1 = jnp.minimum(jnp.maximum(f0 + 1, 0), 127)
    bb = jnp.minimum(c0, 126)
    wc0 = jnp.maximum(1.0 - jnp.abs(r - c0.astype(jnp.float32)), 0.0)
    wc1 = jnp.maximum(1.0 - jnp.abs(r - c1.astype(jnp.float32)), 0.0)
    wa = jnp.where(c0 == bb, wc0, 0.0) + jnp.where(c1 == bb, wc1, 0.0)
    wb = (wc0 + wc1) - wa
    return bb, wa, wb


def _tile_body(tab, params, out,
               idx_s, wgt_s, rows_s, acc_s, pvec, sem0, sem1):
    wid = lax.axis_index("s") * 2 + lax.axis_index("c")
    b = wid // _TPB
    part = wid % _TPB
    pltpu.sync_copy(params.at[b], pvec)

    iota = lax.iota(jnp.int32, 16)
    t00, t01, t02 = pvec[0], pvec[1], pvec[2]
    t10, t11, t12 = pvec[3], pvec[4], pvec[5]
    t20, t21, t22 = pvec[6], pvec[7], pvec[8]
    off = pvec[9]

    vhalf = iota >> 1
    chan = iota & 1
    eoff = b * (_VPB * _C)
    vstart = part * _VPT
    rowbase = part * _NCHUNK

    def phase1(ci, buf):
        row = rowbase + ci
        d = row >> 7
        h = row & 127
        xf = d.astype(jnp.float32) - 64.5
        yf = h.astype(jnp.float32) - 64.5
        basex = t00 * xf + t01 * yf + off
        basey = t10 * xf + t11 * yf + off
        basez = t20 * xf + t21 * yf + off
        for j in range(16):
            lzf = (vhalf + j * 8).astype(jnp.float32) - 64.5
            refx = basex + t02 * lzf
            refy = basey + t12 * lzf
            refz = basez + t22 * lzf
            bx, wxa, wxb = _dim_corners(refx)
            by, wya, wyb = _dim_corners(refy)
            bz, wza, wzb = _dim_corners(refz)
            ex0 = (bx << 15) + (chan + eoff)
            ex1 = ex0 + 32768
            ey0 = by << 8
            ey1 = ey0 + 256
            ez0 = bz << 1
            w00 = wxa * wya
            w01 = wxa * wyb
            w10 = wxb * wya
            w11 = wxb * wyb
            half = j >> 3
            sl = pl.ds((j & 7) * 16, 16)
            for k, (exy, wxy) in enumerate((
                (ex0 + ey0, w00), (ex0 + ey1, w01),
                (ex1 + ey0, w10), (ex1 + ey1, w11))):
                idx_s[buf, 2 * k, half, sl] = exy + ez0
                idx_s[buf, 2 * k + 1, half, sl] = exy + ez0 + 2
                wgt_s[buf, 2 * k, half, sl] = wxy * wza
                wgt_s[buf, 2 * k + 1, half, sl] = wxy * wzb

    def fire(buf, sem):
        for k in range(8):
            for h in range(2):
                pltpu.async_copy(
                    tab.at[idx_s.at[buf].at[k].at[h]],
                    rows_s.at[buf].at[k].at[h], sem)

    def drain(buf, sem):
        for k in range(8):
            for h in range(2):
                pltpu.make_async_copy(
                    tab.at[idx_s.at[buf].at[k].at[h]],
                    rows_s.at[buf].at[k].at[h], sem).wait()

    def phase3(ci, buf):
        for j in range(16):
            half = j >> 3
            sl = pl.ds((j & 7) * 16, 16)
            acc = None
            for k in range(8):
                v = wgt_s[buf, k, half, sl] * rows_s[buf, k, half, sl]
                acc = v if acc is None else acc + v
            acc_s[pl.ds(j * 16, 16)] = acc
        obase = pl.multiple_of(eoff + (vstart + ci * _CH) * _C, _CH * _C)
        pltpu.sync_copy(acc_s, out.at[pl.ds(obase, _CH * _C)])

    phase1(0, 0)
    fire(0, sem0)

    def step(ci2, carry):
        c0 = 2 * ci2
        phase1(c0 + 1, 1)
        fire(1, sem1)
        drain(0, sem0)
        phase3(c0, 0)
        phase1(c0 + 2, 0)
        fire(0, sem0)
        drain(1, sem1)
        phase3(c0 + 1, 1)
        return carry

    lax.fori_loop(0, _NCHUNK // 2, step, 0)
    drain(0, sem0)


@jax.jit
def _grid_sample_sc(tab, params):
    mesh = plsc.VectorSubcoreMesh(core_axis_name="c", subcore_axis_name="s")
    f = pl.kernel(
        _tile_body,
        out_type=jax.ShapeDtypeStruct((_B * _VPB * _C,), jnp.float32),
        mesh=mesh,
        scratch_types=[
            pltpu.VMEM((2, 8, 2, _CH), jnp.int32),
            pltpu.VMEM((2, 8, 2, _CH), jnp.float32),
            pltpu.VMEM((2, 8, 2, _CH), jnp.float32),
            pltpu.VMEM((_CH * _C,), jnp.float32),
            pltpu.VMEM((10, 16), jnp.float32),
            pltpu.SemaphoreType.DMA,
            pltpu.SemaphoreType.DMA,
        ],
        compiler_params=pltpu.CompilerParams(use_tc_tiling_on_sc=False),
    )
    return f(tab, params)


def kernel(images, trans_mates):
    bsz, d, h, w, c = images.shape
    tab = images.reshape(bsz * d * h * w * c)
    eye = jnp.eye(3, dtype=jnp.float32)
    theta = trans_mates[:, :3, :3] * 0.2 + eye
    tu = jax.lax.bitcast_convert_type(theta, jnp.uint32)
    tu = (tu + jnp.uint32(0x7FFF) + ((tu >> 16) & jnp.uint32(1))) \
        & jnp.uint32(0xFFFF0000)
    theta = jax.lax.bitcast_convert_type(tu, jnp.float32)
    off = d * (trans_mates[:, 0, 3] * 0.2 + 0.5) - 0.5
    scal = jnp.concatenate([theta.reshape(bsz, 9), off[:, None]], axis=1)
    params = jnp.broadcast_to(scal[:, :, None], (bsz, 10, 16))
    out = _grid_sample_sc(tab, params)
    return out.reshape(bsz, d, h, w, c)


# speedup vs baseline: 3.9975x; 3.9975x over previous
import functools

import jax
import jax.numpy as jnp
from jax import lax
from jax.experimental import pallas as pl
from jax.experimental.pallas import tpu as pltpu
from jax.experimental.pallas import tpu_sc as plsc

_D = _H = _W = 128
_C = 2
_B = 2
_VPB = _D * _H * _W
_NW = 32
_TPB = 16
_VPT = _VPB // _TPB
_CH = 128
_NCHUNK = _VPT // _CH


def _dim_corners(r):
    ti = r.astype(jnp.int32)
    tf = ti.astype(jnp.float32)
    f0 = ti - jnp.where(tf > r, 1, 0)
    c0 = jnp.minimum(jnp.maximum(f0, 0), 127)
    c1 = jnp.minimum(jnp.maximum(f0 + 1, 0), 127)
    bb = jnp.minimum(c0, 126)
    wc0 = jnp.maximum(1.0 - jnp.abs(r - c0.astype(jnp.float32)), 0.0)
    wc1 = jnp.maximum(1.0 - jnp.abs(r - c1.astype(jnp.float32)), 0.0)
    wa = jnp.where(c0 == bb, wc0, 0.0) + jnp.where(c1 == bb, wc1, 0.0)
    wb = (wc0 + wc1) - wa
    return bb, wa, wb


def _tile_body(tab0, tab1, params, out0, out1,
               idx_s, wgt_s, r0_s, r1_s, a0_s, a1_s, pvec, sem0, sem1):
    wid = lax.axis_index("s") * 2 + lax.axis_index("c")
    b = wid // _TPB
    part = wid % _TPB
    pltpu.sync_copy(params.at[b], pvec)

    iota = lax.iota(jnp.int32, 16)
    t00, t01, t02 = pvec[0], pvec[1], pvec[2]
    t10, t11, t12 = pvec[3], pvec[4], pvec[5]
    t20, t21, t22 = pvec[6], pvec[7], pvec[8]
    off = pvec[9]

    iota_f = iota.astype(jnp.float32)
    boff = b * _VPB
    vstart = part * _VPT
    rowbase = part * _NCHUNK

    def phase1(ci, buf):
        row = rowbase + ci
        d = row >> 7
        h = row & 127
        xf = d.astype(jnp.float32) - 64.5
        yf = h.astype(jnp.float32) - 64.5
        basex = t00 * xf + t01 * yf + off
        basey = t10 * xf + t11 * yf + off
        basez = t20 * xf + t21 * yf + off
        for j in range(8):
            lzf = iota_f + (j * 16 - 64.5)
            refx = basex + t02 * lzf
            refy = basey + t12 * lzf
            refz = basez + t22 * lzf
            bx, wxa, wxb = _dim_corners(refx)
            by, wya, wyb = _dim_corners(refy)
            bz, wza, wzb = _dim_corners(refz)
            rx0 = (bx << 14) + boff
            rx1 = rx0 + 16384
            ry0 = by << 7
            ry1 = ry0 + 128
            w00 = wxa * wya
            w01 = wxa * wyb
            w10 = wxb * wya
            w11 = wxb * wyb
            sl = pl.ds(j * 16, 16)
            for k, (rxy, wxy) in enumerate((
                (rx0 + ry0, w00), (rx0 + ry1, w01),
                (rx1 + ry0, w10), (rx1 + ry1, w11))):
                idx_s[buf, 2 * k, sl] = rxy + bz
                idx_s[buf, 2 * k + 1, sl] = rxy + bz + 1
                wgt_s[buf, 2 * k, sl] = wxy * wza
                wgt_s[buf, 2 * k + 1, sl] = wxy * wzb

    def fire(buf, sem):
        for k in range(8):
            pltpu.async_copy(tab0.at[idx_s.at[buf].at[k]], r0_s.at[buf].at[k], sem)
            pltpu.async_copy(tab1.at[idx_s.at[buf].at[k]], r1_s.at[buf].at[k], sem)

    def drain(buf, sem):
        for k in range(8):
            pltpu.make_async_copy(tab0.at[idx_s.at[buf].at[k]], r0_s.at[buf].at[k], sem).wait()
            pltpu.make_async_copy(tab1.at[idx_s.at[buf].at[k]], r1_s.at[buf].at[k], sem).wait()

    def phase3(ci, buf):
        for j in range(8):
            sl = pl.ds(j * 16, 16)
            a0 = None
            a1 = None
            for k in range(8):
                w = wgt_s[buf, k, sl]
                v0 = w * r0_s[buf, k, sl]
                v1 = w * r1_s[buf, k, sl]
                a0 = v0 if a0 is None else a0 + v0
                a1 = v1 if a1 is None else a1 + v1
            a0_s[sl] = a0
            a1_s[sl] = a1
        obase = pl.multiple_of(boff + vstart + ci * _CH, _CH)
        pltpu.sync_copy(a0_s, out0.at[pl.ds(obase, _CH)])
        pltpu.sync_copy(a1_s, out1.at[pl.ds(obase, _CH)])

    phase1(0, 0)
    fire(0, sem0)

    def step(ci2, carry):
        c0 = 2 * ci2
        phase1(c0 + 1, 1)
        fire(1, sem1)
        drain(0, sem0)
        phase3(c0, 0)
        phase1(c0 + 2, 0)
        fire(0, sem0)
        drain(1, sem1)
        phase3(c0 + 1, 1)
        return carry

    lax.fori_loop(0, _NCHUNK // 2, step, 0)
    drain(0, sem0)


@jax.jit
def _grid_sample_sc(tab0, tab1, params):
    mesh = plsc.VectorSubcoreMesh(core_axis_name="c", subcore_axis_name="s")
    f = pl.kernel(
        _tile_body,
        out_type=[
            jax.ShapeDtypeStruct((_B * _VPB,), jnp.float32),
            jax.ShapeDtypeStruct((_B * _VPB,), jnp.float32),
        ],
        mesh=mesh,
        scratch_types=[
            pltpu.VMEM((2, 8, _CH), jnp.int32),
            pltpu.VMEM((2, 8, _CH), jnp.float32),
            pltpu.VMEM((2, 8, _CH), jnp.float32),
            pltpu.VMEM((2, 8, _CH), jnp.float32),
            pltpu.VMEM((_CH,), jnp.float32),
            pltpu.VMEM((_CH,), jnp.float32),
            pltpu.VMEM((10, 16), jnp.float32),
            pltpu.SemaphoreType.DMA,
            pltpu.SemaphoreType.DMA,
        ],
        compiler_params=pltpu.CompilerParams(use_tc_tiling_on_sc=False),
    )
    return f(tab0, tab1, params)


def kernel(images, trans_mates):
    bsz, d, h, w, c = images.shape
    tabs = images.reshape(bsz * d * h * w, c)
    tab0 = tabs[:, 0]
    tab1 = tabs[:, 1]
    eye = jnp.eye(3, dtype=jnp.float32)
    theta = trans_mates[:, :3, :3] * 0.2 + eye
    tu = jax.lax.bitcast_convert_type(theta, jnp.uint32)
    tu = (tu + jnp.uint32(0x7FFF) + ((tu >> 16) & jnp.uint32(1))) \
        & jnp.uint32(0xFFFF0000)
    theta = jax.lax.bitcast_convert_type(tu, jnp.float32)
    off = d * (trans_mates[:, 0, 3] * 0.2 + 0.5) - 0.5
    scal = jnp.concatenate([theta.reshape(bsz, 9), off[:, None]], axis=1)
    params = jnp.broadcast_to(scal[:, :, None], (bsz, 10, 16))
    o0, o1 = _grid_sample_sc(tab0, tab1, params)
    return jnp.stack([o0, o1], axis=-1).reshape(bsz, d, h, w, c)
